# baseline (speedup 1.0000x reference)
import functools

import jax
import jax.numpy as jnp
from jax.experimental import pallas as pl
from jax.experimental.pallas import tpu as pltpu


def _fused_proj_kernel(x_ref, w1_ref, w2_ref, x1_ref, x2_ref, wcat_ref,
                       *, r_bar):
    @pl.when(pl.program_id(0) == 0)
    def _():
        wcat_ref[:, :r_bar] = w1_ref[...].astype(jnp.bfloat16)
        wcat_ref[:, r_bar:] = jnp.transpose(
            w2_ref[...]).astype(jnp.bfloat16)

    xb = x_ref[...].astype(jnp.bfloat16)
    out = jnp.dot(xb, wcat_ref[...], preferred_element_type=jnp.float32)
    x1_ref[...] = out[:, :r_bar].astype(x1_ref.dtype)
    x2_ref[...] = out[:, r_bar:].astype(x2_ref.dtype)


def kernel(x, dp_mat, vs_weight):
    batch, p = x.shape
    r_bar = dp_mat.shape[1]
    width = vs_weight.shape[0]
    n_out = r_bar + width

    batch_tile = batch
    while batch % batch_tile != 0:
        batch_tile //= 2
    m_steps = batch // batch_tile
    sb = dict(pipeline_mode=pl.Buffered(1))

    grid_spec = pltpu.PrefetchScalarGridSpec(
        num_scalar_prefetch=0,
        grid=(m_steps,),
        in_specs=[
            pl.BlockSpec((batch_tile, p), lambda i: (i, 0), **sb),
            pl.BlockSpec((p, r_bar), lambda i: (0, 0), **sb),
            pl.BlockSpec((width, p), lambda i: (0, 0), **sb),
        ],
        out_specs=[
            pl.BlockSpec((batch_tile, r_bar), lambda i: (i, 0), **sb),
            pl.BlockSpec((batch_tile, width), lambda i: (i, 0), **sb),
        ],
        scratch_shapes=[pltpu.VMEM((p, n_out), jnp.bfloat16)],
    )

    body = functools.partial(_fused_proj_kernel, r_bar=r_bar)

    return pl.pallas_call(
        body,
        out_shape=(
            jax.ShapeDtypeStruct((batch, r_bar), x.dtype),
            jax.ShapeDtypeStruct((batch, width), x.dtype),
        ),
        grid_spec=grid_spec,
        compiler_params=pltpu.CompilerParams(
            dimension_semantics=("arbitrary",),
            vmem_limit_bytes=64 * 1024 * 1024,
        ),
        cost_estimate=pl.CostEstimate(
            flops=2 * batch * p * n_out,
            transcendentals=0,
            bytes_accessed=4 * (batch * p + batch * n_out) + 4 * p * n_out,
        ),
    )(x, dp_mat, vs_weight)


# speedup vs baseline: 1.1470x; 1.1470x over previous
import functools

import jax
import jax.numpy as jnp
from jax.experimental import pallas as pl
from jax.experimental.pallas import tpu as pltpu


def _fused_proj_kernel(x_ref, w1_ref, w2_ref, x1_ref, x2_ref, wcat_ref,
                       *, r_bar):
    @pl.when(pl.program_id(0) == 0)
    def _():
        wcat_ref[:, :r_bar] = w1_ref[...].astype(jnp.bfloat16)
        wcat_ref[:, r_bar:] = jnp.transpose(
            w2_ref[...]).astype(jnp.bfloat16)

    xb = x_ref[...].astype(jnp.bfloat16)
    out = jnp.dot(xb, wcat_ref[...], preferred_element_type=jnp.float32)
    x1_ref[...] = out[:, :r_bar].astype(x1_ref.dtype)
    x2_ref[...] = out[:, r_bar:].astype(x2_ref.dtype)


def kernel(x, dp_mat, vs_weight):
    batch, p = x.shape
    r_bar = dp_mat.shape[1]
    width = vs_weight.shape[0]
    n_out = r_bar + width

    batch_tile = 4096
    while batch % batch_tile != 0:
        batch_tile //= 2
    m_steps = batch // batch_tile
    wb = dict(pipeline_mode=pl.Buffered(1))

    grid_spec = pltpu.PrefetchScalarGridSpec(
        num_scalar_prefetch=0,
        grid=(m_steps,),
        in_specs=[
            pl.BlockSpec((batch_tile, p), lambda i: (i, 0)),
            pl.BlockSpec((p, r_bar), lambda i: (0, 0), **wb),
            pl.BlockSpec((width, p), lambda i: (0, 0), **wb),
        ],
        out_specs=[
            pl.BlockSpec((batch_tile, r_bar), lambda i: (i, 0)),
            pl.BlockSpec((batch_tile, width), lambda i: (i, 0)),
        ],
        scratch_shapes=[pltpu.VMEM((p, n_out), jnp.bfloat16)],
    )

    body = functools.partial(_fused_proj_kernel, r_bar=r_bar)

    return pl.pallas_call(
        body,
        out_shape=(
            jax.ShapeDtypeStruct((batch, r_bar), x.dtype),
            jax.ShapeDtypeStruct((batch, width), x.dtype),
        ),
        grid_spec=grid_spec,
        compiler_params=pltpu.CompilerParams(
            dimension_semantics=("arbitrary",),
            vmem_limit_bytes=64 * 1024 * 1024,
        ),
        cost_estimate=pl.CostEstimate(
            flops=2 * batch * p * n_out,
            transcendentals=0,
            bytes_accessed=4 * (batch * p + batch * n_out) + 4 * p * n_out,
        ),
    )(x, dp_mat, vs_weight)
